# baseline (speedup 1.0000x reference)
import jax
import jax.numpy as jnp
from jax import lax
from jax.experimental import pallas as pl
from jax.experimental.pallas import tpu as pltpu
from jax.experimental.pallas import tpu_sc as plsc

INPUTS = 40960
L1 = 256
B = 16384
_SPLITS = (10240, 6144)

_info = plsc.get_sparse_core_info()
_NC, _NS = _info.num_cores, _info.num_subcores
_NW = _NC * _NS
_CH = 128
_NBUF = 3


def _make_sc_gather_body(bpw, nchunk, ch):
    def body(table_hbm, widx_hbm, bidx_hbm, wout_hbm, bout_hbm,
             widx_v, bidx_v, rows0, rows1, rows2, gsem, wsem, isem):
        wid = lax.axis_index("s") * _NC + lax.axis_index("c")
        base = wid * bpw
        rowsb = (rows0, rows1, rows2)
        widx_cp = pltpu.async_copy(widx_hbm.at[pl.ds(base, bpw)], widx_v,
                                   isem)
        bidx_cp = pltpu.async_copy(bidx_hbm.at[pl.ds(base, bpw)], bidx_v,
                                   isem)
        widx_cp.wait()
        bidx_cp.wait()
        work = []
        for idx_v, dst_hbm in ((widx_v, wout_hbm), (bidx_v, bout_hbm)):
            for c in range(nchunk):
                work.append((idx_v, dst_hbm, c * ch))
        n = len(work)
        lead = _NBUF - 1
        gathers, writebacks = [], []
        for i in range(n + lead):
            if i < n:
                idx_v, dst_hbm, off = work[i]
                if i >= _NBUF:
                    writebacks[i - _NBUF].wait()
                gathers.append(
                    pltpu.async_copy(table_hbm.at[idx_v.at[pl.ds(off, ch)]],
                                     rowsb[i % _NBUF], gsem))
            j = i - lead
            if j >= 0:
                idx_v, dst_hbm, off = work[j]
                gathers[j].wait()
                writebacks.append(
                    pltpu.async_copy(rowsb[j % _NBUF],
                                     dst_hbm.at[pl.ds(base + off, ch)],
                                     wsem))
        for wb in writebacks[-_NBUF:]:
            wb.wait()
    return body


def _sc_gather(table, widx, bidx, nrows):
    bpw = nrows // _NW
    ch = _CH if bpw % _CH == 0 else 64
    nchunk = bpw // ch
    mesh = plsc.VectorSubcoreMesh(core_axis_name="c", subcore_axis_name="s")
    kern = pl.kernel(
        _make_sc_gather_body(bpw, nchunk, ch),
        mesh=mesh,
        out_type=(
            jax.ShapeDtypeStruct((nrows, L1), jnp.float32),
            jax.ShapeDtypeStruct((nrows, L1), jnp.float32),
        ),
        scratch_types=[
            pltpu.VMEM((bpw,), jnp.int32),
            pltpu.VMEM((bpw,), jnp.int32),
            pltpu.VMEM((ch, L1), jnp.float32),
            pltpu.VMEM((ch, L1), jnp.float32),
            pltpu.VMEM((ch, L1), jnp.float32),
            pltpu.SemaphoreType.DMA,
            pltpu.SemaphoreType.DMA,
            pltpu.SemaphoreType.DMA,
        ],
    )
    return kern(table, widx, bidx)


_BLK = 4096
_CONTRACT_MINOR = (((1,), (1,)), ((), ()))


def _make_mlp_body(blk):
    def _mlp_body(wg_ref, bg_ref, stm_ref, fb_ref, w1a_ref, w1b_ref, b1_ref,
                  w2_ref, b2_ref, wo_ref, bo_ref, out_ref):
        s = jnp.reshape(stm_ref[...], (blk, 1))
        w = wg_ref[...]
        b = bg_ref[...]
        fb = fb_ref[...]
        t = s * (w - b)
        xa = jnp.clip(b + fb + t, 0.0, 1.0).astype(jnp.bfloat16)
        xb = jnp.clip(w + fb - t, 0.0, 1.0).astype(jnp.bfloat16)
        h1 = lax.dot_general(xa, w1a_ref[...], _CONTRACT_MINOR,
                             preferred_element_type=jnp.float32)
        h1 += lax.dot_general(xb, w1b_ref[...], _CONTRACT_MINOR,
                              preferred_element_type=jnp.float32)
        h1 = jnp.clip(h1 + b1_ref[...], 0.0, 1.0)
        h2 = jnp.clip(lax.dot_general(h1, w2_ref[...], _CONTRACT_MINOR,
                                      preferred_element_type=jnp.float32)
                      + b2_ref[...], 0.0, 1.0)
        o = jnp.dot(h2, wo_ref[...], preferred_element_type=jnp.float32) \
            + bo_ref[...]
        out_ref[...] = jnp.reshape(o, (1, blk))
    return _mlp_body


def _mlp(wg, bg, stm, fb, w1a16, w1b16, b1, W2, b2, woT, bo, nrows):
    blk = _BLK if nrows % _BLK == 0 else 2048
    grid = nrows // blk
    rows = lambda i: (i, 0)
    cols = lambda i: (0, i)
    rep = lambda i: (0, 0)
    out = pl.pallas_call(
        _make_mlp_body(blk),
        grid=(grid,),
        in_specs=[
            pl.BlockSpec((blk, L1), rows),
            pl.BlockSpec((blk, L1), rows),
            pl.BlockSpec((1, blk), cols),
            pl.BlockSpec((1, L1), rep),
            pl.BlockSpec((32, L1), rep),
            pl.BlockSpec((32, L1), rep),
            pl.BlockSpec((1, 32), rep),
            pl.BlockSpec((32, 32), rep),
            pl.BlockSpec((1, 32), rep),
            pl.BlockSpec((32, 1), rep),
            pl.BlockSpec((1, 1), rep),
        ],
        out_specs=pl.BlockSpec((1, blk), cols),
        out_shape=jax.ShapeDtypeStruct((1, nrows), jnp.float32),
    )(wg, bg, stm.reshape(1, nrows), fb, w1a16, w1b16, b1, W2, b2, woT, bo)
    return out.reshape(nrows)


def kernel(white_features, white_offsets, black_features, black_offsets, stm,
           ft_weight, ft_bias, W1, b1, W2, b2, Wout, bout):
    widx = white_features.astype(jnp.int32)
    bidx = black_features.astype(jnp.int32)
    fb = ft_bias.reshape(1, L1)
    w1a16 = W1[:, :L1].astype(jnp.bfloat16)
    w1b16 = W1[:, L1:].astype(jnp.bfloat16)
    b1r = b1.reshape(1, 32)
    b2r = b2.reshape(1, 32)
    woT = Wout.T
    bo = bout.reshape(1, 1)
    outs = []
    start = 0
    for nb in _SPLITS:
        sl = slice(start, start + nb)
        start += nb
        wg, bg = _sc_gather(ft_weight, widx[sl], bidx[sl], nb)
        outs.append(_mlp(wg, bg, stm[sl], fb, w1a16, w1b16, b1r, W2, b2r,
                         woT, bo, nb))
    return jnp.concatenate(outs)


# speedup vs baseline: 1.0322x; 1.0322x over previous
import jax
import jax.numpy as jnp
from jax import lax
from jax.experimental import pallas as pl
from jax.experimental.pallas import tpu as pltpu
from jax.experimental.pallas import tpu_sc as plsc

INPUTS = 40960
L1 = 256
B = 16384
_SPLITS = (8192, 8192)

_info = plsc.get_sparse_core_info()
_NC, _NS = _info.num_cores, _info.num_subcores
_NW = _NC * _NS
_CH = 128
_NBUF = 3


def _make_sc_gather_body(bpw, nchunk, ch):
    def body(table_hbm, widx_hbm, bidx_hbm, wout_hbm, bout_hbm,
             widx_v, bidx_v, rows0, rows1, rows2, gsem, wsem, isem):
        wid = lax.axis_index("s") * _NC + lax.axis_index("c")
        base = wid * bpw
        rowsb = (rows0, rows1, rows2)
        widx_cp = pltpu.async_copy(widx_hbm.at[pl.ds(base, bpw)], widx_v,
                                   isem)
        bidx_cp = pltpu.async_copy(bidx_hbm.at[pl.ds(base, bpw)], bidx_v,
                                   isem)
        widx_cp.wait()
        bidx_cp.wait()
        work = []
        for idx_v, dst_hbm in ((widx_v, wout_hbm), (bidx_v, bout_hbm)):
            for c in range(nchunk):
                work.append((idx_v, dst_hbm, c * ch))
        n = len(work)
        lead = _NBUF - 1
        gathers, writebacks = [], []
        for i in range(n + lead):
            if i < n:
                idx_v, dst_hbm, off = work[i]
                if i >= _NBUF:
                    writebacks[i - _NBUF].wait()
                gathers.append(
                    pltpu.async_copy(table_hbm.at[idx_v.at[pl.ds(off, ch)]],
                                     rowsb[i % _NBUF], gsem))
            j = i - lead
            if j >= 0:
                idx_v, dst_hbm, off = work[j]
                gathers[j].wait()
                writebacks.append(
                    pltpu.async_copy(rowsb[j % _NBUF],
                                     dst_hbm.at[pl.ds(base + off, ch)],
                                     wsem))
        for wb in writebacks[-_NBUF:]:
            wb.wait()
    return body


def _sc_gather(table, widx, bidx, nrows):
    bpw = nrows // _NW
    ch = _CH if bpw % _CH == 0 else 64
    nchunk = bpw // ch
    mesh = plsc.VectorSubcoreMesh(core_axis_name="c", subcore_axis_name="s")
    kern = pl.kernel(
        _make_sc_gather_body(bpw, nchunk, ch),
        mesh=mesh,
        out_type=(
            jax.ShapeDtypeStruct((nrows, L1), jnp.float32),
            jax.ShapeDtypeStruct((nrows, L1), jnp.float32),
        ),
        scratch_types=[
            pltpu.VMEM((bpw,), jnp.int32),
            pltpu.VMEM((bpw,), jnp.int32),
            pltpu.VMEM((ch, L1), jnp.float32),
            pltpu.VMEM((ch, L1), jnp.float32),
            pltpu.VMEM((ch, L1), jnp.float32),
            pltpu.SemaphoreType.DMA,
            pltpu.SemaphoreType.DMA,
            pltpu.SemaphoreType.DMA,
        ],
    )
    return kern(table, widx, bidx)


_BLK = 4096
_CONTRACT_MINOR = (((1,), (1,)), ((), ()))


def _make_mlp_body(blk):
    def _mlp_body(wg_ref, bg_ref, stm_ref, fb_ref, w1a_ref, w1b_ref, b1_ref,
                  w2_ref, b2_ref, wo_ref, bo_ref, out_ref):
        s = jnp.reshape(stm_ref[...], (blk, 1))
        w = wg_ref[...]
        b = bg_ref[...]
        fb = fb_ref[...]
        t = s * (w - b)
        xa = jnp.clip(b + fb + t, 0.0, 1.0).astype(jnp.bfloat16)
        xb = jnp.clip(w + fb - t, 0.0, 1.0).astype(jnp.bfloat16)
        h1 = lax.dot_general(xa, w1a_ref[...], _CONTRACT_MINOR,
                             preferred_element_type=jnp.float32)
        h1 += lax.dot_general(xb, w1b_ref[...], _CONTRACT_MINOR,
                              preferred_element_type=jnp.float32)
        h1 = jnp.clip(h1 + b1_ref[...], 0.0, 1.0)
        h2 = jnp.clip(lax.dot_general(h1, w2_ref[...], _CONTRACT_MINOR,
                                      preferred_element_type=jnp.float32)
                      + b2_ref[...], 0.0, 1.0)
        o = jnp.dot(h2, wo_ref[...], preferred_element_type=jnp.float32) \
            + bo_ref[...]
        out_ref[...] = jnp.reshape(o, (1, blk))
    return _mlp_body


def _mlp(wg, bg, stm, fb, w1a16, w1b16, b1, W2, b2, woT, bo, nrows):
    blk = _BLK if nrows % _BLK == 0 else 2048
    grid = nrows // blk
    rows = lambda i: (i, 0)
    cols = lambda i: (0, i)
    rep = lambda i: (0, 0)
    out = pl.pallas_call(
        _make_mlp_body(blk),
        grid=(grid,),
        in_specs=[
            pl.BlockSpec((blk, L1), rows),
            pl.BlockSpec((blk, L1), rows),
            pl.BlockSpec((1, blk), cols),
            pl.BlockSpec((1, L1), rep),
            pl.BlockSpec((32, L1), rep),
            pl.BlockSpec((32, L1), rep),
            pl.BlockSpec((1, 32), rep),
            pl.BlockSpec((32, 32), rep),
            pl.BlockSpec((1, 32), rep),
            pl.BlockSpec((32, 1), rep),
            pl.BlockSpec((1, 1), rep),
        ],
        out_specs=pl.BlockSpec((1, blk), cols),
        out_shape=jax.ShapeDtypeStruct((1, nrows), jnp.float32),
    )(wg, bg, stm.reshape(1, nrows), fb, w1a16, w1b16, b1, W2, b2, woT, bo)
    return out.reshape(nrows)


def kernel(white_features, white_offsets, black_features, black_offsets, stm,
           ft_weight, ft_bias, W1, b1, W2, b2, Wout, bout):
    widx = white_features.astype(jnp.int32)
    bidx = black_features.astype(jnp.int32)
    fb = ft_bias.reshape(1, L1)
    w1a16 = W1[:, :L1].astype(jnp.bfloat16)
    w1b16 = W1[:, L1:].astype(jnp.bfloat16)
    b1r = b1.reshape(1, 32)
    b2r = b2.reshape(1, 32)
    woT = Wout.T
    bo = bout.reshape(1, 1)
    outs = []
    start = 0
    for nb in _SPLITS:
        sl = slice(start, start + nb)
        start += nb
        wg, bg = _sc_gather(ft_weight, widx[sl], bidx[sl], nb)
        outs.append(_mlp(wg, bg, stm[sl], fb, w1a16, w1b16, b1r, W2, b2r,
                         woT, bo, nb))
    return jnp.concatenate(outs)
